# Initial kernel scaffold; baseline (speedup 1.0000x reference)
import jax
import jax.numpy as jnp
from jax.experimental import pallas as pl


def kernel(edge_src, edge_dst, node_features, edge_sh, edge_length_embedded, num_neighbors, W1, W2):
    raise NotImplementedError("write your pallas kernel here")


# speedup vs baseline: 1.1843x; 1.1843x over previous
import functools

import jax
import jax.numpy as jnp
import numpy as np
from jax import lax
from jax.experimental import pallas as pl
from jax.experimental.pallas import tpu as pltpu
from jax.experimental.pallas import tpu_sc as plsc

N_NODES = 10000
D_IN = 16
D_OUT = 16
HIDDEN = 256

_NC = 2
_NS = 16


def _sc_gather(table, idx, chunk=2000):
    E = idx.shape[0]
    D = table.shape[1]
    nw = _NC * _NS
    per_w = E // nw
    n_ch = per_w // chunk
    mesh = plsc.VectorSubcoreMesh(core_axis_name="c", subcore_axis_name="s")

    @functools.partial(
        pl.kernel,
        mesh=mesh,
        out_type=jax.ShapeDtypeStruct((E, D), jnp.float32),
        scratch_types=[
            pltpu.VMEM((chunk,), jnp.int32),
            pltpu.VMEM((chunk, D), jnp.float32),
            pltpu.SemaphoreType.DMA,
        ],
        compiler_params=pltpu.CompilerParams(use_tc_tiling_on_sc=False),
    )
    def k(table_hbm, idx_hbm, out_hbm, idx_v, rows_v, sem):
        wid = lax.axis_index("s") * _NC + lax.axis_index("c")
        base = wid * per_w

        def body(i, carry):
            off = base + i * chunk
            pltpu.sync_copy(idx_hbm.at[pl.ds(off, chunk)], idx_v)
            pltpu.async_copy(table_hbm.at[idx_v], rows_v, sem).wait()
            pltpu.sync_copy(rows_v, out_hbm.at[pl.ds(off, chunk)])
            return carry

        lax.fori_loop(0, n_ch, body, 0)

    return k(table, idx)


def _sc_scatter_add(rows, idx, n_out, chunk=2000):
    E, D = rows.shape
    per_core = E // _NC
    per_w = per_core // _NS
    n_ch = per_w // chunk
    rows_per_tile = n_out // _NS
    mesh = plsc.VectorSubcoreMesh(core_axis_name="c", subcore_axis_name="s")

    @functools.partial(
        pl.kernel,
        mesh=mesh,
        out_type=jax.ShapeDtypeStruct((_NC, n_out, D), jnp.float32),
        scratch_types=[
            pltpu.VMEM((chunk,), jnp.int32),
            pltpu.VMEM((chunk, D), jnp.float32),
            pltpu.VMEM_SHARED((n_out, D), jnp.float32),
            pltpu.SemaphoreType.DMA,
        ],
        compiler_params=pltpu.CompilerParams(use_tc_tiling_on_sc=False),
    )
    def k(rows_hbm, idx_hbm, zeros_hbm, out_hbm, idx_v, rows_v, accum, sem):
        c = lax.axis_index("c")
        s = lax.axis_index("s")
        base = c * per_core + s * per_w

        zoff = s * rows_per_tile
        pltpu.sync_copy(
            zeros_hbm.at[pl.ds(zoff, rows_per_tile)],
            accum.at[pl.ds(zoff, rows_per_tile)],
        )
        plsc.subcore_barrier()

        def body(i, carry):
            off = base + i * chunk
            pltpu.sync_copy(idx_hbm.at[pl.ds(off, chunk)], idx_v)
            pltpu.sync_copy(rows_hbm.at[pl.ds(off, chunk)], rows_v)
            pltpu.sync_copy(rows_v, accum.at[idx_v], add=True)
            return carry

        lax.fori_loop(0, n_ch, body, 0)
        plsc.subcore_barrier()

        pltpu.sync_copy(
            accum.at[pl.ds(zoff, rows_per_tile)],
            out_hbm.at[c, pl.ds(zoff, rows_per_tile)],
        )

    zeros = jnp.zeros((n_out, D), jnp.float32)
    return k(rows, idx, zeros)


def _tc_dense(lpad, sh, x_src, w1n, w2n, block=2000):
    E = x_src.shape[0]
    grid = E // block

    def body(l_ref, sh_ref, x_ref, w1_ref, w2_ref, o_ref):
        h = jnp.maximum(
            jnp.dot(l_ref[...], w1_ref[...], preferred_element_type=jnp.float32), 0.0
        )
        wt = jnp.dot(h, w2_ref[...], preferred_element_type=jnp.float32)
        x = x_ref[...]
        acc = x[:, 0:1] * wt[:, 0:D_OUT]
        for i in range(1, D_IN):
            acc = acc + x[:, i : i + 1] * wt[:, i * D_OUT : (i + 1) * D_OUT]
        o_ref[...] = acc * sh_ref[...]

    return pl.pallas_call(
        body,
        grid=(grid,),
        in_specs=[
            pl.BlockSpec((block, 8), lambda i: (i, 0)),
            pl.BlockSpec((block, 1), lambda i: (i, 0)),
            pl.BlockSpec((block, D_IN), lambda i: (i, 0)),
            pl.BlockSpec((8, HIDDEN), lambda i: (0, 0)),
            pl.BlockSpec((HIDDEN, HIDDEN), lambda i: (0, 0)),
        ],
        out_specs=pl.BlockSpec((block, D_OUT), lambda i: (i, 0)),
        out_shape=jax.ShapeDtypeStruct((E, D_OUT), jnp.float32),
    )(lpad, sh, x_src, w1n, w2n)


def _tc_combine(partials):
    def body(p_ref, o_ref):
        o_ref[...] = p_ref[0] + p_ref[1]

    n, d = partials.shape[1], partials.shape[2]
    return pl.pallas_call(
        body,
        out_shape=jax.ShapeDtypeStruct((n, d), jnp.float32),
    )(partials)


def kernel(edge_src, edge_dst, node_features, edge_sh, edge_length_embedded,
           num_neighbors, W1, W2):
    E = edge_src.shape[0]
    w1n = (W1 * np.float32(1.0 / np.sqrt(3.0))).astype(jnp.float32)
    scale = np.float32(np.sqrt(2.0) / np.sqrt(float(HIDDEN)) / np.sqrt(float(D_IN)))
    w2n = W2 * (scale / jnp.sqrt(jnp.float32(num_neighbors)))
    lpad = jnp.pad(edge_length_embedded, ((0, 0), (0, 5)))
    w1p = jnp.pad(w1n, ((0, 5), (0, 0)))

    x_src = _sc_gather(node_features, edge_src.astype(jnp.int32))
    ef = _tc_dense(lpad, edge_sh, x_src, w1p, w2n)
    partials = _sc_scatter_add(ef, edge_dst.astype(jnp.int32), N_NODES)
    return _tc_combine(partials)


# speedup vs baseline: 1.2517x; 1.0569x over previous
import functools

import jax
import jax.numpy as jnp
import numpy as np
from jax import lax
from jax.experimental import pallas as pl
from jax.experimental.pallas import tpu as pltpu
from jax.experimental.pallas import tpu_sc as plsc

N_NODES = 10000
D_IN = 16
D_OUT = 16
HIDDEN = 256

_NC = 2
_NS = 16
_SCP = pltpu.CompilerParams(use_tc_tiling_on_sc=False)


def _sc_gather(table, idx, chunk=2000):
    E = idx.shape[0]
    D = D_IN
    nw = _NC * _NS
    per_w = E // nw
    n_ch = per_w // chunk
    ckr = chunk * D // 128
    mesh = plsc.VectorSubcoreMesh(core_axis_name="c", subcore_axis_name="s")

    @functools.partial(
        pl.kernel,
        mesh=mesh,
        out_type=jax.ShapeDtypeStruct((E * D // 128, 128), jnp.float32),
        scratch_types=[
            pltpu.VMEM((chunk,), jnp.int32),
            pltpu.VMEM((chunk, D), jnp.float32),
            pltpu.VMEM((ckr, 128), jnp.float32),
            pltpu.SemaphoreType.DMA,
        ],
        compiler_params=_SCP,
    )
    def k(table_hbm, idx_hbm, out_hbm, idx_v, rows_v, packed_v, sem):
        wid = lax.axis_index("s") * _NC + lax.axis_index("c")
        base = wid * per_w

        def body(i, carry):
            off = base + i * chunk
            pltpu.sync_copy(idx_hbm.at[pl.ds(off, chunk)], idx_v)
            pltpu.async_copy(table_hbm.at[idx_v], rows_v, sem).wait()

            def pack(j, c2):
                for l in range(8):
                    packed_v[j, pl.ds(l * D, D)] = rows_v[j * 8 + l, :]
                return c2

            lax.fori_loop(0, ckr, pack, 0)
            pltpu.sync_copy(packed_v, out_hbm.at[pl.ds((off * D) // 128, ckr)])
            return carry

        lax.fori_loop(0, n_ch, body, 0)

    return k(table, idx)


def _sc_scatter_add(rows_packed, idx, n_out, chunk=2000):
    D = D_OUT
    E = rows_packed.shape[0] * 128 // D
    per_core = E // _NC
    per_w = per_core // _NS
    n_ch = per_w // chunk
    ckr = chunk * D // 128
    rpt = ((n_out // _NS) // 8) * 8
    rem = n_out - rpt * _NS
    out_rows = n_out * D // 128
    mesh = plsc.VectorSubcoreMesh(core_axis_name="c", subcore_axis_name="s")

    @functools.partial(
        pl.kernel,
        mesh=mesh,
        out_type=jax.ShapeDtypeStruct((_NC, out_rows, 128), jnp.float32),
        scratch_types=[
            pltpu.VMEM((chunk,), jnp.int32),
            pltpu.VMEM((ckr, 128), jnp.float32),
            pltpu.VMEM((chunk, D), jnp.float32),
            pltpu.VMEM((rpt + rem, D), jnp.float32),
            pltpu.VMEM(((rpt + rem) * D // 128, 128), jnp.float32),
            pltpu.VMEM_SHARED((n_out, D), jnp.float32),
            pltpu.SemaphoreType.DMA,
        ],
        compiler_params=_SCP,
    )
    def k(rp_hbm, idx_hbm, zeros_hbm, out_hbm, idx_v, packed_v, rows_v, bounce,
          bpk, accum, sem):
        c = lax.axis_index("c")
        s = lax.axis_index("s")
        base = c * per_core + s * per_w
        zoff = s * rpt

        pltpu.sync_copy(
            zeros_hbm.at[pl.ds(zoff, rpt)],
            accum.at[pl.ds(zoff, rpt)],
        )

        @pl.when(s == _NS - 1)
        def _zero_tail():
            pltpu.sync_copy(
                zeros_hbm.at[pl.ds(rpt * _NS, rem)],
                accum.at[pl.ds(rpt * _NS, rem)],
            )

        plsc.subcore_barrier()

        def body(i, carry):
            off = base + i * chunk
            pltpu.sync_copy(idx_hbm.at[pl.ds(off, chunk)], idx_v)
            pltpu.sync_copy(rp_hbm.at[pl.ds((off * D) // 128, ckr)], packed_v)

            def unpack(j, c2):
                for l in range(8):
                    rows_v[j * 8 + l, :] = packed_v[j, pl.ds(l * D, D)]
                return c2

            lax.fori_loop(0, ckr, unpack, 0)
            pltpu.sync_copy(rows_v, accum.at[idx_v], add=True)
            return carry

        lax.fori_loop(0, n_ch, body, 0)
        plsc.subcore_barrier()

        def flush(src_off, nrows, dst_row):
            pltpu.sync_copy(accum.at[pl.ds(src_off, nrows)], bounce.at[pl.ds(0, nrows)])

            def pack(j, c2):
                for l in range(8):
                    bpk[j, pl.ds(l * D, D)] = bounce[j * 8 + l, :]
                return c2

            lax.fori_loop(0, nrows * D // 128, pack, 0)
            pltpu.sync_copy(
                bpk.at[pl.ds(0, nrows * D // 128)],
                out_hbm.at[c, pl.ds(dst_row, nrows * D // 128)],
            )

        flush(zoff, rpt, (zoff * D) // 128)

        @pl.when(s == _NS - 1)
        def _tail():
            flush(rpt * _NS, rem, (rpt * _NS * D) // 128)

    zeros = jnp.zeros((n_out, D), jnp.float32)
    return k(rows_packed, idx, zeros)


def _tc_dense(ell, sh, xp, w1n, w2n, block=3200):
    E = ell.shape[0]
    grid = E // block
    pk = block * D_IN // 128

    sub = block // 8

    def body(l_ref, sh_ref, x_ref, w1_ref, w2_ref, o_ref):
        h = jnp.maximum(
            jnp.dot(l_ref[...], w1_ref[...], preferred_element_type=jnp.float32), 0.0
        )
        wt = jnp.dot(h, w2_ref[...], preferred_element_type=jnp.float32)
        xp = x_ref[...]
        x = jnp.concatenate(
            [xp[:, D_IN * l : D_IN * (l + 1)] for l in range(8)], axis=0
        )
        acc = x[:, 0:1] * wt[:, 0:D_OUT]
        for i in range(1, D_IN):
            acc = acc + x[:, i : i + 1] * wt[:, i * D_OUT : (i + 1) * D_OUT]
        accsh = acc * sh_ref[...]
        o_ref[...] = jnp.concatenate(
            [accsh[sub * l : sub * (l + 1), :] for l in range(8)], axis=1
        )

    return pl.pallas_call(
        body,
        grid=(grid,),
        in_specs=[
            pl.BlockSpec((block, 3), lambda i: (i, 0)),
            pl.BlockSpec((block, 1), lambda i: (i, 0)),
            pl.BlockSpec((pk, 128), lambda i: (i, 0)),
            pl.BlockSpec((3, HIDDEN), lambda i: (0, 0)),
            pl.BlockSpec((HIDDEN, HIDDEN), lambda i: (0, 0)),
        ],
        out_specs=pl.BlockSpec((pk, 128), lambda i: (i, 0)),
        out_shape=jax.ShapeDtypeStruct((E * D_OUT // 128, 128), jnp.float32),
    )(ell, sh, xp, w1n, w2n)


def _tc_combine(partials):
    def body(p_ref, o_ref):
        o_ref[...] = p_ref[0] + p_ref[1]

    n, d = partials.shape[1], partials.shape[2]
    return pl.pallas_call(
        body,
        out_shape=jax.ShapeDtypeStruct((n, d), jnp.float32),
    )(partials)


def kernel(edge_src, edge_dst, node_features, edge_sh, edge_length_embedded,
           num_neighbors, W1, W2):
    E = edge_src.shape[0]
    w1n = (W1 * np.float32(1.0 / np.sqrt(3.0))).astype(jnp.float32)
    scale = np.float32(np.sqrt(2.0) / np.sqrt(float(HIDDEN)) / np.sqrt(float(D_IN)))
    w2n = W2 * (scale / jnp.sqrt(jnp.float32(num_neighbors)))

    blk = 3200

    def to_tc_order(a):
        c = a.shape[-1]
        return a.reshape(-1, blk // 8, 8, c).transpose(0, 2, 1, 3).reshape(E, c)

    xp = _sc_gather(node_features, edge_src.astype(jnp.int32))
    efp = _tc_dense(
        to_tc_order(edge_length_embedded), to_tc_order(edge_sh), xp, w1n, w2n,
        block=blk,
    )
    partials = _sc_scatter_add(efp, edge_dst.astype(jnp.int32), N_NODES)
    return _tc_combine(partials).reshape(N_NODES, D_OUT)


# speedup vs baseline: 1.8588x; 1.4850x over previous
import functools

import jax
import jax.numpy as jnp
import numpy as np
from jax import lax
from jax.experimental import pallas as pl
from jax.experimental.pallas import tpu as pltpu
from jax.experimental.pallas import tpu_sc as plsc

N_NODES = 10000
D_IN = 16
D_OUT = 16
HIDDEN = 256

_NC = 2
_NS = 16
_SCP = pltpu.CompilerParams(use_tc_tiling_on_sc=False)


def _sc_gather(table, idx, chunk=2000):
    E = idx.shape[0]
    D = D_IN
    nw = _NC * _NS
    per_w = E // nw
    n_ch = per_w // chunk
    ckr = chunk * D // 128
    mesh = plsc.VectorSubcoreMesh(core_axis_name="c", subcore_axis_name="s")

    @functools.partial(
        pl.kernel,
        mesh=mesh,
        out_type=jax.ShapeDtypeStruct((E * D // 128, 128), jnp.float32),
        scratch_types=[
            pltpu.VMEM((chunk,), jnp.int32),
            pltpu.VMEM((chunk, D), jnp.float32),
            pltpu.VMEM((ckr, 128), jnp.float32),
            pltpu.SemaphoreType.DMA,
        ],
        compiler_params=_SCP,
    )
    def k(table_hbm, idx_hbm, out_hbm, idx_v, rows_v, packed_v, sem):
        wid = lax.axis_index("s") * _NC + lax.axis_index("c")
        base = wid * per_w

        def body(i, carry):
            off = base + i * chunk
            pltpu.sync_copy(idx_hbm.at[pl.ds(off, chunk)], idx_v)
            pltpu.async_copy(table_hbm.at[idx_v], rows_v, sem).wait()

            def pack(j, c2):
                for l in range(8):
                    packed_v[j, pl.ds(l * D, D)] = rows_v[l * ckr + j, :]
                return c2

            lax.fori_loop(0, ckr, pack, 0)
            pltpu.sync_copy(packed_v, out_hbm.at[pl.ds((off * D) // 128, ckr)])
            return carry

        lax.fori_loop(0, n_ch, body, 0)

    return k(table, idx)


def _sc_scatter_add(rows_packed, idx, n_out, chunk=2000):
    D = D_OUT
    E = rows_packed.shape[0] * 128 // D
    per_core = E // _NC
    per_w = per_core // _NS
    n_ch = per_w // chunk
    ckr = chunk * D // 128
    rpt = ((n_out // _NS) // 8) * 8
    rem = n_out - rpt * _NS
    out_rows = n_out * D // 128
    mesh = plsc.VectorSubcoreMesh(core_axis_name="c", subcore_axis_name="s")

    @functools.partial(
        pl.kernel,
        mesh=mesh,
        out_type=jax.ShapeDtypeStruct((_NC, out_rows, 128), jnp.float32),
        scratch_types=[
            pltpu.VMEM((chunk,), jnp.int32),
            pltpu.VMEM((ckr, 128), jnp.float32),
            pltpu.VMEM((chunk, D), jnp.float32),
            pltpu.VMEM((rpt + rem, D), jnp.float32),
            pltpu.VMEM(((rpt + rem) * D // 128, 128), jnp.float32),
            pltpu.VMEM_SHARED((n_out, D), jnp.float32),
            pltpu.SemaphoreType.DMA,
        ],
        compiler_params=_SCP,
    )
    def k(rp_hbm, idx_hbm, zeros_hbm, out_hbm, idx_v, packed_v, rows_v, bounce,
          bpk, accum, sem):
        c = lax.axis_index("c")
        s = lax.axis_index("s")
        base = c * per_core + s * per_w
        zoff = s * rpt

        pltpu.sync_copy(
            zeros_hbm.at[pl.ds(zoff, rpt)],
            accum.at[pl.ds(zoff, rpt)],
        )

        @pl.when(s == _NS - 1)
        def _zero_tail():
            pltpu.sync_copy(
                zeros_hbm.at[pl.ds(rpt * _NS, rem)],
                accum.at[pl.ds(rpt * _NS, rem)],
            )

        plsc.subcore_barrier()

        def body(i, carry):
            off = base + i * chunk
            pltpu.sync_copy(idx_hbm.at[pl.ds(off, chunk)], idx_v)
            pltpu.sync_copy(rp_hbm.at[pl.ds((off * D) // 128, ckr)], packed_v)

            def unpack(j, c2):
                for l in range(8):
                    rows_v[l * ckr + j, :] = packed_v[j, pl.ds(l * D, D)]
                return c2

            lax.fori_loop(0, ckr, unpack, 0)
            pltpu.sync_copy(rows_v, accum.at[idx_v], add=True)
            return carry

        lax.fori_loop(0, n_ch, body, 0)
        plsc.subcore_barrier()

        def flush(src_off, nrows, dst_row):
            pltpu.sync_copy(accum.at[pl.ds(src_off, nrows)], bounce.at[pl.ds(0, nrows)])

            def pack(j, c2):
                for l in range(8):
                    bpk[j, pl.ds(l * D, D)] = bounce[j * 8 + l, :]
                return c2

            lax.fori_loop(0, nrows * D // 128, pack, 0)
            pltpu.sync_copy(
                bpk.at[pl.ds(0, nrows * D // 128)],
                out_hbm.at[c, pl.ds(dst_row, nrows * D // 128)],
            )

        flush(zoff, rpt, (zoff * D) // 128)

        @pl.when(s == _NS - 1)
        def _tail():
            flush(rpt * _NS, rem, (rpt * _NS * D) // 128)

    zeros = jnp.zeros((n_out, D), jnp.float32)
    return k(rows_packed, idx, zeros)


def _tc_dense(ell, sh, xp, w1n, w2n, block=2000):
    E = ell.shape[0]
    grid = E // block
    pk = block * D_IN // 128
    sub = block // 8

    def body(l_ref, sh_ref, x_ref, w1_ref, w2_ref, o_ref):
        h = jnp.maximum(
            jnp.dot(l_ref[...], w1_ref[...], preferred_element_type=jnp.float32), 0.0
        )
        wt = jnp.dot(h, w2_ref[...], preferred_element_type=jnp.float32)
        xpb = x_ref[0]
        x = jnp.concatenate(
            [xpb[:, D_IN * l : D_IN * (l + 1)] for l in range(8)], axis=0
        )
        acc = x[:, 0:1] * wt[:, 0:D_OUT]
        for i in range(1, D_IN):
            acc = acc + x[:, i : i + 1] * wt[:, i * D_OUT : (i + 1) * D_OUT]
        accsh = acc * sh_ref[...]
        o_ref[0] = jnp.concatenate(
            [accsh[sub * l : sub * (l + 1), :] for l in range(8)], axis=1
        )

    return pl.pallas_call(
        body,
        grid=(grid,),
        in_specs=[
            pl.BlockSpec((block, 3), lambda i: (i, 0)),
            pl.BlockSpec((block, 1), lambda i: (i, 0)),
            pl.BlockSpec((1, pk, 128), lambda i: (i, 0, 0)),
            pl.BlockSpec((3, HIDDEN), lambda i: (0, 0)),
            pl.BlockSpec((HIDDEN, HIDDEN), lambda i: (0, 0)),
        ],
        out_specs=pl.BlockSpec((1, pk, 128), lambda i: (i, 0, 0)),
        out_shape=jax.ShapeDtypeStruct((grid, pk, 128), jnp.float32),
    )(ell, sh, xp.reshape(grid, pk, 128), w1n, w2n).reshape(E * D_OUT // 128, 128)


def _tc_combine(partials):
    def body(p_ref, o_ref):
        o_ref[...] = p_ref[0] + p_ref[1]

    n, d = partials.shape[1], partials.shape[2]
    return pl.pallas_call(
        body,
        out_shape=jax.ShapeDtypeStruct((n, d), jnp.float32),
    )(partials)


def kernel(edge_src, edge_dst, node_features, edge_sh, edge_length_embedded,
           num_neighbors, W1, W2):
    E = edge_src.shape[0]
    w1n = (W1 * np.float32(1.0 / np.sqrt(3.0))).astype(jnp.float32)
    scale = np.float32(np.sqrt(2.0) / np.sqrt(float(HIDDEN)) / np.sqrt(float(D_IN)))
    w2n = W2 * (scale / jnp.sqrt(jnp.float32(num_neighbors)))

    xp = _sc_gather(node_features, edge_src.astype(jnp.int32))
    efp = _tc_dense(edge_length_embedded, edge_sh, xp, w1n, w2n)
    partials = _sc_scatter_add(efp, edge_dst.astype(jnp.int32), N_NODES)
    return _tc_combine(partials).reshape(N_NODES, D_OUT)


# speedup vs baseline: 6.1364x; 3.3013x over previous
import functools

import jax
import jax.numpy as jnp
import numpy as np
from jax import lax
from jax.experimental import pallas as pl
from jax.experimental.pallas import tpu as pltpu
from jax.experimental.pallas import tpu_sc as plsc

N_NODES = 10000
D_IN = 16
D_OUT = 16
HIDDEN = 256

_NC = 2
_NS = 16
_SCP = pltpu.CompilerParams(use_tc_tiling_on_sc=False)


def _sc_gather(table, idx, chunk=2000):
    E = idx.shape[0]
    D = D_IN
    nw = _NC * _NS
    per_w = E // nw
    n_ch = per_w // chunk
    ckr = chunk * D // 128
    mesh = plsc.VectorSubcoreMesh(core_axis_name="c", subcore_axis_name="s")

    @functools.partial(
        pl.kernel,
        mesh=mesh,
        out_type=jax.ShapeDtypeStruct((E * D // 128, 128), jnp.float32),
        scratch_types=[
            pltpu.VMEM((chunk,), jnp.int32),
            pltpu.VMEM((chunk, D), jnp.float32),
            pltpu.VMEM((ckr, 128), jnp.float32),
            pltpu.SemaphoreType.DMA,
        ],
        compiler_params=_SCP,
    )
    def k(table_hbm, idx_hbm, out_hbm, idx_v, rows_v, packed_v, sem):
        wid = lax.axis_index("s") * _NC + lax.axis_index("c")
        base = wid * per_w

        def body(i, carry):
            off = base + i * chunk
            pltpu.sync_copy(idx_hbm.at[pl.ds(off, chunk)], idx_v)
            pltpu.async_copy(table_hbm.at[idx_v], rows_v, sem).wait()

            def pack(j, c2):
                for l in range(8):
                    packed_v[j, pl.ds(l * D, D)] = rows_v[l * ckr + j, :]
                return c2

            lax.fori_loop(0, ckr, pack, 0)
            pltpu.sync_copy(packed_v, out_hbm.at[pl.ds((off * D) // 128, ckr)])
            return carry

        lax.fori_loop(0, n_ch, body, 0)

    return k(table, idx)


def _sc_scatter_add(rows_packed, idx, n_out, chunk=2000):
    D = D_OUT
    E = rows_packed.shape[0] * 128 // D
    per_core = E // _NC
    per_w = per_core // _NS
    n_ch = per_w // chunk
    ckr = chunk * D // 128
    rpt = ((n_out // _NS) // 8) * 8
    rem = n_out - rpt * _NS
    out_rows = n_out * D // 128
    mesh = plsc.VectorSubcoreMesh(core_axis_name="c", subcore_axis_name="s")

    @functools.partial(
        pl.kernel,
        mesh=mesh,
        out_type=jax.ShapeDtypeStruct((_NC, out_rows, 128), jnp.float32),
        scratch_types=[
            pltpu.VMEM((chunk,), jnp.int32),
            pltpu.VMEM((ckr, 128), jnp.float32),
            pltpu.VMEM((chunk, D), jnp.float32),
            pltpu.VMEM((rpt + rem, D), jnp.float32),
            pltpu.VMEM(((rpt + rem) * D // 128, 128), jnp.float32),
            pltpu.VMEM_SHARED((n_out, D), jnp.float32),
            pltpu.SemaphoreType.DMA,
        ],
        compiler_params=_SCP,
    )
    def k(rp_hbm, idx_hbm, zeros_hbm, out_hbm, idx_v, packed_v, rows_v, bounce,
          bpk, accum, sem):
        c = lax.axis_index("c")
        s = lax.axis_index("s")
        base = c * per_core + s * per_w
        zoff = s * rpt

        pltpu.sync_copy(
            zeros_hbm.at[pl.ds(zoff, rpt)],
            accum.at[pl.ds(zoff, rpt)],
        )

        @pl.when(s == _NS - 1)
        def _zero_tail():
            pltpu.sync_copy(
                zeros_hbm.at[pl.ds(rpt * _NS, rem)],
                accum.at[pl.ds(rpt * _NS, rem)],
            )

        plsc.subcore_barrier()

        def body(i, carry):
            off = base + i * chunk
            pltpu.sync_copy(idx_hbm.at[pl.ds(off, chunk)], idx_v)
            pltpu.sync_copy(rp_hbm.at[pl.ds((off * D) // 128, ckr)], packed_v)

            def unpack(j, c2):
                for l in range(8):
                    rows_v[l * ckr + j, :] = packed_v[j, pl.ds(l * D, D)]
                return c2

            lax.fori_loop(0, ckr, unpack, 0)
            pltpu.sync_copy(rows_v, accum.at[idx_v], add=True)
            return carry

        lax.fori_loop(0, n_ch, body, 0)
        plsc.subcore_barrier()

        def flush(src_off, nrows, dst_row):
            pltpu.sync_copy(accum.at[pl.ds(src_off, nrows)], bounce.at[pl.ds(0, nrows)])

            def pack(j, c2):
                for l in range(8):
                    bpk[j, pl.ds(l * D, D)] = bounce[j * 8 + l, :]
                return c2

            lax.fori_loop(0, nrows * D // 128, pack, 0)
            pltpu.sync_copy(
                bpk.at[pl.ds(0, nrows * D // 128)],
                out_hbm.at[c, pl.ds(dst_row, nrows * D // 128)],
            )

        flush(zoff, rpt, (zoff * D) // 128)

        @pl.when(s == _NS - 1)
        def _tail():
            flush(rpt * _NS, rem, (rpt * _NS * D) // 128)

    zeros = jnp.zeros((n_out, D), jnp.float32)
    return k(rows_packed, idx, zeros)


def _tc_dense(ell, sh, xp, w1n, w2n, block=2000):
    E = ell.shape[0]
    grid = E // block
    pk = block * D_IN // 128
    sub = block // 8

    R = np.zeros((8, 128, HIDDEN), np.float32)
    U = np.zeros((8, HIDDEN, 128), np.float32)
    for l in range(8):
        for i in range(D_IN):
            for k in range(D_OUT):
                R[l, 16 * l + i, 16 * i + k] = 1.0
                U[l, 16 * i + k, 16 * l + k] = 1.0
    rcat = jnp.asarray(R.reshape(8 * 128, HIDDEN))
    ucat = jnp.asarray(U.reshape(8 * HIDDEN, 128))

    def body(l_ref, sh_ref, x_ref, w1_ref, w2_ref, r_ref, u_ref, o_ref):
        h = jnp.maximum(
            jnp.dot(l_ref[...], w1_ref[...], preferred_element_type=jnp.float32), 0.0
        )
        wt = jnp.dot(h, w2_ref[...], preferred_element_type=jnp.float32)
        wts = wt * sh_ref[...]
        xpb = x_ref[0]
        o = None
        for l in range(8):
            xr = jnp.dot(
                xpb, r_ref[128 * l : 128 * (l + 1), :],
                preferred_element_type=jnp.float32,
            )
            y = xr * wts[sub * l : sub * (l + 1), :]
            t = jnp.dot(
                y, u_ref[HIDDEN * l : HIDDEN * (l + 1), :],
                preferred_element_type=jnp.float32,
            )
            o = t if o is None else o + t
        o_ref[0] = o

    return pl.pallas_call(
        body,
        grid=(grid,),
        in_specs=[
            pl.BlockSpec((block, 3), lambda i: (i, 0)),
            pl.BlockSpec((block, 1), lambda i: (i, 0)),
            pl.BlockSpec((1, pk, 128), lambda i: (i, 0, 0)),
            pl.BlockSpec((3, HIDDEN), lambda i: (0, 0)),
            pl.BlockSpec((HIDDEN, HIDDEN), lambda i: (0, 0)),
            pl.BlockSpec((8 * 128, HIDDEN), lambda i: (0, 0)),
            pl.BlockSpec((8 * HIDDEN, 128), lambda i: (0, 0)),
        ],
        out_specs=pl.BlockSpec((1, pk, 128), lambda i: (i, 0, 0)),
        out_shape=jax.ShapeDtypeStruct((grid, pk, 128), jnp.float32),
    )(ell, sh, xp.reshape(grid, pk, 128), w1n, w2n, rcat, ucat).reshape(
        E * D_OUT // 128, 128
    )


def _tc_combine(partials):
    def body(p_ref, o_ref):
        o_ref[...] = p_ref[0] + p_ref[1]

    n, d = partials.shape[1], partials.shape[2]
    return pl.pallas_call(
        body,
        out_shape=jax.ShapeDtypeStruct((n, d), jnp.float32),
    )(partials)


def kernel(edge_src, edge_dst, node_features, edge_sh, edge_length_embedded,
           num_neighbors, W1, W2):
    E = edge_src.shape[0]
    w1n = (W1 * np.float32(1.0 / np.sqrt(3.0))).astype(jnp.float32)
    scale = np.float32(np.sqrt(2.0) / np.sqrt(float(HIDDEN)) / np.sqrt(float(D_IN)))
    w2n = W2 * (scale / jnp.sqrt(jnp.float32(num_neighbors)))

    xp = _sc_gather(node_features, edge_src.astype(jnp.int32))
    efp = _tc_dense(edge_length_embedded, edge_sh, xp, w1n, w2n)
    partials = _sc_scatter_add(efp, edge_dst.astype(jnp.int32), N_NODES)
    return _tc_combine(partials).reshape(N_NODES, D_OUT)


# speedup vs baseline: 7.2098x; 1.1749x over previous
import functools

import jax
import jax.numpy as jnp
import numpy as np
from jax import lax
from jax.experimental import pallas as pl
from jax.experimental.pallas import tpu as pltpu
from jax.experimental.pallas import tpu_sc as plsc

N_NODES = 10000
D_IN = 16
D_OUT = 16
HIDDEN = 256

_NC = 2
_NS = 16
_SCP = pltpu.CompilerParams(use_tc_tiling_on_sc=False)


def _sc_gather(table, idx, chunk=2000):
    E = idx.shape[0]
    D = D_IN
    nw = _NC * _NS
    per_w = E // nw
    n_ch = per_w // chunk
    ckr = chunk * D // 128
    mesh = plsc.VectorSubcoreMesh(core_axis_name="c", subcore_axis_name="s")

    @functools.partial(
        pl.kernel,
        mesh=mesh,
        out_type=jax.ShapeDtypeStruct((E * D // 128, 128), jnp.float32),
        scratch_types=[
            pltpu.VMEM((chunk,), jnp.int32),
            pltpu.VMEM((chunk, D), jnp.float32),
            pltpu.VMEM((ckr, 128), jnp.float32),
            pltpu.SemaphoreType.DMA,
        ],
        compiler_params=_SCP,
    )
    def k(table_hbm, idx_hbm, out_hbm, idx_v, rows_v, packed_v, sem):
        wid = lax.axis_index("s") * _NC + lax.axis_index("c")
        base = wid * per_w

        def body(i, carry):
            off = base + i * chunk
            pltpu.sync_copy(idx_hbm.at[pl.ds(off, chunk)], idx_v)
            pltpu.async_copy(table_hbm.at[idx_v], rows_v, sem).wait()

            def pack(j, c2):
                for l in range(8):
                    packed_v[j, pl.ds(l * D, D)] = rows_v[l * ckr + j, :]
                return c2

            lax.fori_loop(0, ckr, pack, 0)
            pltpu.sync_copy(packed_v, out_hbm.at[pl.ds((off * D) // 128, ckr)])
            return carry

        lax.fori_loop(0, n_ch, body, 0)

    return k(table, idx)


def _sc_scatter_add(rows_packed, idx, n_out, chunk=2000):
    D = D_OUT
    E = rows_packed.shape[0] * 128 // D
    per_core = E // _NC
    per_w = per_core // _NS
    n_ch = per_w // chunk
    ckr = chunk * D // 128
    rpt = ((n_out // _NS) // 8) * 8
    rem = n_out - rpt * _NS
    out_rows = n_out * D // 128
    mesh = plsc.VectorSubcoreMesh(core_axis_name="c", subcore_axis_name="s")

    @functools.partial(
        pl.kernel,
        mesh=mesh,
        out_type=jax.ShapeDtypeStruct((_NC, out_rows, 128), jnp.float32),
        scratch_types=[
            pltpu.VMEM((chunk,), jnp.int32),
            pltpu.VMEM((ckr, 128), jnp.float32),
            pltpu.VMEM((chunk, D), jnp.float32),
            pltpu.VMEM((rpt + rem, D), jnp.float32),
            pltpu.VMEM(((rpt + rem) * D // 128, 128), jnp.float32),
            pltpu.VMEM_SHARED((n_out, D), jnp.float32),
            pltpu.SemaphoreType.DMA,
        ],
        compiler_params=_SCP,
    )
    def k(rp_hbm, idx_hbm, zeros_hbm, out_hbm, idx_v, packed_v, rows_v, bounce,
          bpk, accum, sem):
        c = lax.axis_index("c")
        s = lax.axis_index("s")
        base = c * per_core + s * per_w
        zoff = s * rpt

        pltpu.sync_copy(
            zeros_hbm.at[pl.ds(zoff, rpt)],
            accum.at[pl.ds(zoff, rpt)],
        )

        @pl.when(s == _NS - 1)
        def _zero_tail():
            pltpu.sync_copy(
                zeros_hbm.at[pl.ds(rpt * _NS, rem)],
                accum.at[pl.ds(rpt * _NS, rem)],
            )

        plsc.subcore_barrier()

        def body(i, carry):
            off = base + i * chunk
            pltpu.sync_copy(idx_hbm.at[pl.ds(off, chunk)], idx_v)
            pltpu.sync_copy(rp_hbm.at[pl.ds((off * D) // 128, ckr)], packed_v)

            def unpack(j, c2):
                for l in range(8):
                    rows_v[l * ckr + j, :] = packed_v[j, pl.ds(l * D, D)]
                return c2

            lax.fori_loop(0, ckr, unpack, 0)
            pltpu.sync_copy(rows_v, accum.at[idx_v], add=True)
            return carry

        lax.fori_loop(0, n_ch, body, 0)
        plsc.subcore_barrier()

        def flush(src_off, nrows, dst_row):
            pltpu.sync_copy(accum.at[pl.ds(src_off, nrows)], bounce.at[pl.ds(0, nrows)])

            def pack(j, c2):
                for l in range(8):
                    bpk[j, pl.ds(l * D, D)] = bounce[j * 8 + l, :]
                return c2

            lax.fori_loop(0, nrows * D // 128, pack, 0)
            pltpu.sync_copy(
                bpk.at[pl.ds(0, nrows * D // 128)],
                out_hbm.at[c, pl.ds(dst_row, nrows * D // 128)],
            )

        flush(zoff, rpt, (zoff * D) // 128)

        @pl.when(s == _NS - 1)
        def _tail():
            flush(rpt * _NS, rem, (rpt * _NS * D) // 128)

    zeros = jnp.zeros((n_out, D), jnp.float32)
    return k(rows_packed, idx, zeros)


def _tc_dense(ell, xp, w1n, w2n, block=2000):
    E = ell.shape[0]
    grid = E // block
    pk = block * D_IN // 128
    sub = block // 8

    R = np.zeros((8, 128, HIDDEN), np.float32)
    U = np.zeros((8, HIDDEN, 128), np.float32)
    for l in range(8):
        for i in range(D_IN):
            for k in range(D_OUT):
                R[l, 16 * l + i, 16 * i + k] = 1.0
                U[l, 16 * i + k, 16 * l + k] = 1.0
    rcat = jnp.asarray(R.reshape(8 * 128, HIDDEN))
    ucat = jnp.asarray(U.reshape(8 * HIDDEN, 128))

    def body(lsh_ref, x_ref, w1_ref, w2_ref, r_ref, u_ref, o_ref):
        lsh = lsh_ref[...]
        h = jnp.maximum(
            jnp.dot(lsh, w1_ref[...], preferred_element_type=jnp.float32), 0.0
        )
        wt = jnp.dot(h, w2_ref[...], preferred_element_type=jnp.float32)
        wts = wt * lsh[:, 3:4]
        xpb = x_ref[0]
        o = None
        for l in range(8):
            xr = jnp.dot(
                xpb, r_ref[128 * l : 128 * (l + 1), :],
                preferred_element_type=jnp.float32,
            )
            y = xr * wts[sub * l : sub * (l + 1), :]
            t = jnp.dot(
                y, u_ref[HIDDEN * l : HIDDEN * (l + 1), :],
                preferred_element_type=jnp.float32,
            )
            o = t if o is None else o + t
        o_ref[0] = o

    return pl.pallas_call(
        body,
        grid=(grid,),
        in_specs=[
            pl.BlockSpec((block, 4), lambda i: (i, 0)),
            pl.BlockSpec((1, pk, 128), lambda i: (i, 0, 0)),
            pl.BlockSpec((4, HIDDEN), lambda i: (0, 0)),
            pl.BlockSpec((HIDDEN, HIDDEN), lambda i: (0, 0)),
            pl.BlockSpec((8 * 128, HIDDEN), lambda i: (0, 0)),
            pl.BlockSpec((8 * HIDDEN, 128), lambda i: (0, 0)),
        ],
        out_specs=pl.BlockSpec((1, pk, 128), lambda i: (i, 0, 0)),
        out_shape=jax.ShapeDtypeStruct((grid, pk, 128), jnp.float32),
    )(ell, xp.reshape(grid, pk, 128), w1n, w2n, rcat, ucat).reshape(
        E * D_OUT // 128, 128
    )


def _tc_combine(partials):
    def body(p_ref, o_ref):
        o_ref[...] = p_ref[0] + p_ref[1]

    n, d = partials.shape[1], partials.shape[2]
    return pl.pallas_call(
        body,
        out_shape=jax.ShapeDtypeStruct((n, d), jnp.float32),
    )(partials)


def kernel(edge_src, edge_dst, node_features, edge_sh, edge_length_embedded,
           num_neighbors, W1, W2):
    E = edge_src.shape[0]
    w1n = (W1 * np.float32(1.0 / np.sqrt(3.0))).astype(jnp.float32)
    scale = np.float32(np.sqrt(2.0) / np.sqrt(float(HIDDEN)) / np.sqrt(float(D_IN)))
    w2n = W2 * (scale / jnp.sqrt(jnp.float32(num_neighbors)))

    lsh = jnp.concatenate([edge_length_embedded, edge_sh], axis=1)
    w1p = jnp.pad(w1n, ((0, 1), (0, 0)))

    xp = _sc_gather(node_features, edge_src.astype(jnp.int32))
    efp = _tc_dense(lsh, xp, w1p, w2n)
    partials = _sc_scatter_add(efp, edge_dst.astype(jnp.int32), N_NODES)
    return _tc_combine(partials).reshape(N_NODES, D_OUT)


# speedup vs baseline: 7.6050x; 1.0548x over previous
import functools

import jax
import jax.numpy as jnp
import numpy as np
from jax import lax
from jax.experimental import pallas as pl
from jax.experimental.pallas import tpu as pltpu
from jax.experimental.pallas import tpu_sc as plsc

N_NODES = 10000
D_IN = 16
D_OUT = 16
HIDDEN = 256

_NC = 2
_NS = 16
_SCP = pltpu.CompilerParams(use_tc_tiling_on_sc=False)


def _sc_gather(table, idx, chunk=2000):
    E = idx.shape[0]
    D = D_IN
    nw = _NC * _NS
    per_w = E // nw
    n_ch = per_w // chunk
    ckr = chunk * D // 128
    mesh = plsc.VectorSubcoreMesh(core_axis_name="c", subcore_axis_name="s")

    @functools.partial(
        pl.kernel,
        mesh=mesh,
        out_type=jax.ShapeDtypeStruct((E * D // 128, 128), jnp.float32),
        scratch_types=[
            pltpu.VMEM((chunk,), jnp.int32),
            pltpu.VMEM((chunk, D), jnp.float32),
            pltpu.VMEM((ckr, 128), jnp.float32),
            pltpu.SemaphoreType.DMA,
        ],
        compiler_params=_SCP,
    )
    def k(table_hbm, idx_hbm, out_hbm, idx_v, rows_v, packed_v, sem):
        wid = lax.axis_index("s") * _NC + lax.axis_index("c")
        base = wid * per_w

        def body(i, carry):
            off = base + i * chunk
            pltpu.sync_copy(idx_hbm.at[pl.ds(off, chunk)], idx_v)
            pltpu.async_copy(table_hbm.at[idx_v], rows_v, sem).wait()

            def pack(j, c2):
                for l in range(8):
                    packed_v[j, pl.ds(l * D, D)] = rows_v[l * ckr + j, :]
                return c2

            lax.fori_loop(0, ckr, pack, 0)
            pltpu.sync_copy(packed_v, out_hbm.at[pl.ds((off * D) // 128, ckr)])
            return carry

        lax.fori_loop(0, n_ch, body, 0)

    return k(table, idx)


def _sc_scatter_add(rows_packed, idx, n_out, chunk=2000):
    D = D_OUT
    E = rows_packed.shape[0] * 128 // D
    per_core = E // _NC
    per_w = per_core // _NS
    n_ch = per_w // chunk
    ckr = chunk * D // 128
    rpt = ((n_out // _NS) // 8) * 8
    rem = n_out - rpt * _NS
    out_rows = n_out * D // 128
    mesh = plsc.VectorSubcoreMesh(core_axis_name="c", subcore_axis_name="s")

    @functools.partial(
        pl.kernel,
        mesh=mesh,
        out_type=jax.ShapeDtypeStruct((_NC, out_rows, 128), jnp.float32),
        scratch_types=[
            pltpu.VMEM((chunk,), jnp.int32),
            pltpu.VMEM((ckr, 128), jnp.float32),
            pltpu.VMEM((chunk, D), jnp.float32),
            pltpu.VMEM((rpt + rem, D), jnp.float32),
            pltpu.VMEM(((rpt + rem) * D // 128, 128), jnp.float32),
            pltpu.VMEM_SHARED((n_out, D), jnp.float32),
            pltpu.SemaphoreType.DMA,
        ],
        compiler_params=_SCP,
    )
    def k(rp_hbm, idx_hbm, zeros_hbm, out_hbm, idx_v, packed_v, rows_v, bounce,
          bpk, accum, sem):
        c = lax.axis_index("c")
        s = lax.axis_index("s")
        base = c * per_core + s * per_w
        zoff = s * rpt

        pltpu.sync_copy(
            zeros_hbm.at[pl.ds(zoff, rpt)],
            accum.at[pl.ds(zoff, rpt)],
        )

        @pl.when(s == _NS - 1)
        def _zero_tail():
            pltpu.sync_copy(
                zeros_hbm.at[pl.ds(rpt * _NS, rem)],
                accum.at[pl.ds(rpt * _NS, rem)],
            )

        plsc.subcore_barrier()

        def body(i, carry):
            off = base + i * chunk
            pltpu.sync_copy(idx_hbm.at[pl.ds(off, chunk)], idx_v)
            pltpu.sync_copy(rp_hbm.at[pl.ds((off * D) // 128, ckr)], packed_v)

            def unpack(j, c2):
                for l in range(8):
                    rows_v[l * ckr + j, :] = packed_v[j, pl.ds(l * D, D)]
                return c2

            lax.fori_loop(0, ckr, unpack, 0)
            pltpu.sync_copy(rows_v, accum.at[idx_v], add=True)
            return carry

        lax.fori_loop(0, n_ch, body, 0)
        plsc.subcore_barrier()

        def flush(src_off, nrows, dst_row):
            pltpu.sync_copy(accum.at[pl.ds(src_off, nrows)], bounce.at[pl.ds(0, nrows)])

            def pack(j, c2):
                for l in range(8):
                    bpk[j, pl.ds(l * D, D)] = bounce[j * 8 + l, :]
                return c2

            lax.fori_loop(0, nrows * D // 128, pack, 0)
            pltpu.sync_copy(
                bpk.at[pl.ds(0, nrows * D // 128)],
                out_hbm.at[c, pl.ds(dst_row, nrows * D // 128)],
            )

        flush(zoff, rpt, (zoff * D) // 128)

        @pl.when(s == _NS - 1)
        def _tail():
            flush(rpt * _NS, rem, (rpt * _NS * D) // 128)

    zeros = jnp.zeros((n_out, D), jnp.float32)
    return k(rows_packed, idx, zeros)


def _tc_dense(ell, xp, w1n, w2n, block=2000):
    E = ell.shape[0]
    grid = E // block
    pk = block * D_IN // 128
    sub = block // 8

    R = np.zeros((8, 128, HIDDEN), np.float32)
    U = np.zeros((8, HIDDEN, 128), np.float32)
    for l in range(8):
        for i in range(D_IN):
            for k in range(D_OUT):
                R[l, 16 * l + i, 16 * i + k] = 1.0
                U[l, 16 * i + k, 16 * l + k] = 1.0
    rcat = jnp.asarray(R.reshape(8 * 128, HIDDEN))
    ucat = jnp.asarray(U.reshape(8 * HIDDEN, 128))

    def body(lsh_ref, x_ref, w1_ref, w2_ref, r_ref, u_ref, o_ref):
        lsh = lsh_ref[...]
        h = jnp.maximum(
            jnp.dot(lsh, w1_ref[...], preferred_element_type=jnp.float32), 0.0
        )
        wt = jnp.dot(h, w2_ref[...], preferred_element_type=jnp.float32)
        wts = wt * lsh[:, 3:4]
        xpb = x_ref[0]
        o = None
        for l in range(8):
            xr = jnp.dot(
                xpb, r_ref[128 * l : 128 * (l + 1), :],
                preferred_element_type=jnp.float32,
            )
            y = xr * wts[sub * l : sub * (l + 1), :]
            t = jnp.dot(
                y, u_ref[HIDDEN * l : HIDDEN * (l + 1), :],
                preferred_element_type=jnp.float32,
            )
            o = t if o is None else o + t
        o_ref[0] = o

    return pl.pallas_call(
        body,
        grid=(grid,),
        in_specs=[
            pl.BlockSpec((block, 4), lambda i: (i, 0)),
            pl.BlockSpec((1, pk, 128), lambda i: (i, 0, 0)),
            pl.BlockSpec((4, HIDDEN), lambda i: (0, 0)),
            pl.BlockSpec((HIDDEN, HIDDEN), lambda i: (0, 0)),
            pl.BlockSpec((8 * 128, HIDDEN), lambda i: (0, 0)),
            pl.BlockSpec((8 * HIDDEN, 128), lambda i: (0, 0)),
        ],
        out_specs=pl.BlockSpec((1, pk, 128), lambda i: (i, 0, 0)),
        out_shape=jax.ShapeDtypeStruct((grid, pk, 128), jnp.float32),
    )(ell, xp.reshape(grid, pk, 128), w1n, w2n, rcat, ucat).reshape(
        E * D_OUT // 128, 128
    )


def _tc_combine(pa, pb):
    def body(a_ref, b_ref, o_ref):
        o_ref[...] = a_ref[0] + a_ref[1] + b_ref[0] + b_ref[1]

    n, d = pa.shape[1], pa.shape[2]
    return pl.pallas_call(
        body,
        out_shape=jax.ShapeDtypeStruct((n, d), jnp.float32),
    )(pa, pb)


def kernel(edge_src, edge_dst, node_features, edge_sh, edge_length_embedded,
           num_neighbors, W1, W2):
    E = edge_src.shape[0]
    w1n = (W1 * np.float32(1.0 / np.sqrt(3.0))).astype(jnp.float32)
    scale = np.float32(np.sqrt(2.0) / np.sqrt(float(HIDDEN)) / np.sqrt(float(D_IN)))
    w2n = W2 * (scale / jnp.sqrt(jnp.float32(num_neighbors)))

    lsh = jnp.concatenate([edge_length_embedded, edge_sh], axis=1)
    w1p = jnp.pad(w1n, ((0, 1), (0, 0)))

    Eh = E // 2
    src = edge_src.astype(jnp.int32)
    dst = edge_dst.astype(jnp.int32)
    parts = []
    efps = []
    for hf in range(2):
        sl = slice(hf * Eh, (hf + 1) * Eh)
        xp = _sc_gather(node_features, src[sl])
        efps.append(_tc_dense(lsh[sl], xp, w1p, w2n))
    for hf in range(2):
        sl = slice(hf * Eh, (hf + 1) * Eh)
        parts.append(_sc_scatter_add(efps[hf], dst[sl], N_NODES))
    return _tc_combine(*parts).reshape(N_NODES, D_OUT)


# speedup vs baseline: 8.1221x; 1.0680x over previous
import functools

import jax
import jax.numpy as jnp
import numpy as np
from jax import lax
from jax.experimental import pallas as pl
from jax.experimental.pallas import tpu as pltpu
from jax.experimental.pallas import tpu_sc as plsc

N_NODES = 10000
D_IN = 16
D_OUT = 16
HIDDEN = 256

_NC = 2
_NS = 16
_SCP = pltpu.CompilerParams(use_tc_tiling_on_sc=False)


def _sc_gather(table, idx, chunk=2000):
    E = idx.shape[0]
    D = D_IN
    nw = _NC * _NS
    per_w = E // nw
    n_ch = per_w // chunk
    ckr = chunk * D // 128
    mesh = plsc.VectorSubcoreMesh(core_axis_name="c", subcore_axis_name="s")

    @functools.partial(
        pl.kernel,
        mesh=mesh,
        out_type=jax.ShapeDtypeStruct((E // chunk, ckr, 128), jnp.float32),
        scratch_types=[
            pltpu.VMEM((chunk,), jnp.int32),
            pltpu.VMEM((chunk, D), jnp.float32),
            pltpu.VMEM((ckr, 128), jnp.float32),
            pltpu.SemaphoreType.DMA,
        ],
        compiler_params=_SCP,
    )
    def k(table_hbm, idx_hbm, out_hbm, idx_v, rows_v, packed_v, sem):
        wid = lax.axis_index("s") * _NC + lax.axis_index("c")
        base = wid * per_w

        def body(i, carry):
            off = base + i * chunk
            pltpu.sync_copy(idx_hbm.at[pl.ds(off, chunk)], idx_v)
            pltpu.async_copy(table_hbm.at[idx_v], rows_v, sem).wait()

            def pack(j, c2):
                for l in range(8):
                    packed_v[j, pl.ds(l * D, D)] = rows_v[l * ckr + j, :]
                return c2

            lax.fori_loop(0, ckr, pack, 0)
            pltpu.sync_copy(packed_v, out_hbm.at[off // chunk])
            return carry

        lax.fori_loop(0, n_ch, body, 0)

    return k(table, idx)


def _sc_scatter_add(rows_packed, idx, n_out, chunk=2000):
    D = D_OUT
    E = rows_packed.shape[0] * chunk
    per_core = E // _NC
    per_w = per_core // _NS
    n_ch = per_w // chunk
    ckr = chunk * D // 128
    rpt = ((n_out // _NS) // 8) * 8
    rem = n_out - rpt * _NS
    out_rows = 1280
    mesh = plsc.VectorSubcoreMesh(core_axis_name="c", subcore_axis_name="s")

    @functools.partial(
        pl.kernel,
        mesh=mesh,
        out_type=jax.ShapeDtypeStruct((_NC, out_rows, 128), jnp.float32),
        scratch_types=[
            pltpu.VMEM((chunk,), jnp.int32),
            pltpu.VMEM((ckr, 128), jnp.float32),
            pltpu.VMEM((chunk, D), jnp.float32),
            pltpu.VMEM((rpt + rem, D), jnp.float32),
            pltpu.VMEM(((rpt + rem) * D // 128, 128), jnp.float32),
            pltpu.VMEM_SHARED((n_out, D), jnp.float32),
            pltpu.SemaphoreType.DMA,
        ],
        compiler_params=_SCP,
    )
    def k(rp_hbm, idx_hbm, zeros_hbm, out_hbm, idx_v, packed_v, rows_v, bounce,
          bpk, accum, sem):
        c = lax.axis_index("c")
        s = lax.axis_index("s")
        base = c * per_core + s * per_w
        zoff = s * rpt

        pltpu.sync_copy(
            zeros_hbm.at[pl.ds(zoff, rpt)],
            accum.at[pl.ds(zoff, rpt)],
        )

        @pl.when(s == _NS - 1)
        def _zero_tail():
            pltpu.sync_copy(
                zeros_hbm.at[pl.ds(rpt * _NS, rem)],
                accum.at[pl.ds(rpt * _NS, rem)],
            )

        plsc.subcore_barrier()

        def body(i, carry):
            off = base + i * chunk
            pltpu.sync_copy(idx_hbm.at[pl.ds(off, chunk)], idx_v)
            pltpu.sync_copy(rp_hbm.at[off // chunk], packed_v)

            def unpack(j, c2):
                for l in range(8):
                    rows_v[l * ckr + j, :] = packed_v[j, pl.ds(l * D, D)]
                return c2

            lax.fori_loop(0, ckr, unpack, 0)
            pltpu.sync_copy(rows_v, accum.at[idx_v], add=True)
            return carry

        lax.fori_loop(0, n_ch, body, 0)
        plsc.subcore_barrier()

        def flush(src_off, nrows, dst_row):
            pltpu.sync_copy(accum.at[pl.ds(src_off, nrows)], bounce.at[pl.ds(0, nrows)])

            def pack(j, c2):
                for l in range(8):
                    bpk[j, pl.ds(l * D, D)] = bounce[j * 8 + l, :]
                return c2

            lax.fori_loop(0, nrows * D // 128, pack, 0)
            pltpu.sync_copy(
                bpk.at[pl.ds(0, nrows * D // 128)],
                out_hbm.at[c, pl.ds(dst_row, nrows * D // 128)],
            )

        flush(zoff, rpt, (zoff * D) // 128)

        @pl.when(s == _NS - 1)
        def _tail():
            flush(rpt * _NS, rem, (rpt * _NS * D) // 128)

    zeros = jnp.zeros((n_out, D), jnp.float32)
    return k(rows_packed, idx, zeros)


def _tc_dense(ell, xp, w1n, w2n, block=2000):
    E = ell.shape[0]
    grid = E // block
    pk = block * D_IN // 128
    sub = block // 8
    assert xp.shape == (grid, pk, 128)

    R = np.zeros((8, 128, HIDDEN), np.float32)
    U = np.zeros((8, HIDDEN, 128), np.float32)
    for l in range(8):
        for i in range(D_IN):
            for k in range(D_OUT):
                R[l, 16 * l + i, 16 * i + k] = 1.0
                U[l, 16 * i + k, 16 * l + k] = 1.0
    rcat = jnp.asarray(R.reshape(8 * 128, HIDDEN))
    ucat = jnp.asarray(U.reshape(8 * HIDDEN, 128))

    def body(lsh_ref, x_ref, w1_ref, w2_ref, r_ref, u_ref, o_ref):
        lsh = lsh_ref[...].astype(jnp.float32)
        h = jnp.maximum(
            jnp.dot(lsh, w1_ref[...], preferred_element_type=jnp.float32), 0.0
        )
        wt = jnp.dot(h, w2_ref[...], preferred_element_type=jnp.float32)
        wts = wt * lsh[:, 3:4]
        xpb = x_ref[0]
        o = None
        for l in range(8):
            xr = jnp.dot(
                xpb, r_ref[128 * l : 128 * (l + 1), :],
                preferred_element_type=jnp.float32,
            )
            y = xr * wts[sub * l : sub * (l + 1), :]
            t = jnp.dot(
                y, u_ref[HIDDEN * l : HIDDEN * (l + 1), :],
                preferred_element_type=jnp.float32,
            )
            o = t if o is None else o + t
        o_ref[0] = o

    return pl.pallas_call(
        body,
        grid=(grid,),
        in_specs=[
            pl.BlockSpec((block, 4), lambda i: (i, 0)),
            pl.BlockSpec((1, pk, 128), lambda i: (i, 0, 0)),
            pl.BlockSpec((4, HIDDEN), lambda i: (0, 0)),
            pl.BlockSpec((HIDDEN, HIDDEN), lambda i: (0, 0)),
            pl.BlockSpec((8 * 128, HIDDEN), lambda i: (0, 0)),
            pl.BlockSpec((8 * HIDDEN, 128), lambda i: (0, 0)),
        ],
        out_specs=pl.BlockSpec((1, pk, 128), lambda i: (i, 0, 0)),
        out_shape=jax.ShapeDtypeStruct((grid, pk, 128), jnp.float32),
    )(ell, xp, w1n, w2n, rcat, ucat)


def _tc_combine(pa, pb):
    rows = pa.shape[1]
    blk = 128

    def body(a_ref, b_ref, o_ref):
        o_ref[...] = a_ref[0] + a_ref[1] + b_ref[0] + b_ref[1]

    return pl.pallas_call(
        body,
        grid=(rows // blk,),
        in_specs=[
            pl.BlockSpec((2, blk, 128), lambda i: (0, i, 0)),
            pl.BlockSpec((2, blk, 128), lambda i: (0, i, 0)),
        ],
        out_specs=pl.BlockSpec((blk, 128), lambda i: (i, 0)),
        out_shape=jax.ShapeDtypeStruct((rows, 128), jnp.float32),
    )(pa, pb)


def kernel(edge_src, edge_dst, node_features, edge_sh, edge_length_embedded,
           num_neighbors, W1, W2):
    E = edge_src.shape[0]
    w1n = (W1 * np.float32(1.0 / np.sqrt(3.0))).astype(jnp.float32)
    scale = np.float32(np.sqrt(2.0) / np.sqrt(float(HIDDEN)) / np.sqrt(float(D_IN)))
    w2n = W2 * (scale / jnp.sqrt(jnp.float32(num_neighbors)))

    lsh = jnp.concatenate(
        [edge_length_embedded, edge_sh], axis=1
    ).astype(jnp.bfloat16)
    w1p = jnp.pad(w1n, ((0, 1), (0, 0)))

    Eh = E // 2
    src = edge_src.astype(jnp.int32)
    dst = edge_dst.astype(jnp.int32)
    parts = []
    efps = []
    for hf in range(2):
        sl = slice(hf * Eh, (hf + 1) * Eh)
        xp = _sc_gather(node_features, src[sl])
        efps.append(_tc_dense(lsh[sl], xp, w1p, w2n))
    for hf in range(2):
        sl = slice(hf * Eh, (hf + 1) * Eh)
        parts.append(_sc_scatter_add(efps[hf], dst[sl], N_NODES))
    comb = _tc_combine(*parts)
    return comb.reshape(-1)[: N_NODES * D_OUT].reshape(N_NODES, D_OUT)


# speedup vs baseline: 8.1780x; 1.0069x over previous
import functools

import jax
import jax.numpy as jnp
import numpy as np
from jax import lax
from jax.experimental import pallas as pl
from jax.experimental.pallas import tpu as pltpu
from jax.experimental.pallas import tpu_sc as plsc

N_NODES = 10000
D_IN = 16
D_OUT = 16
HIDDEN = 256

_NC = 2
_NS = 16
_SCP = pltpu.CompilerParams(use_tc_tiling_on_sc=False)


def _sc_gather(table, idx, base_edge, n_edges, chunk=2000):
    D = D_IN
    nw = _NC * _NS
    per_w = n_edges // nw
    n_ch = per_w // chunk
    ckr = chunk * D // 128
    mesh = plsc.VectorSubcoreMesh(core_axis_name="c", subcore_axis_name="s")

    @functools.partial(
        pl.kernel,
        mesh=mesh,
        out_type=jax.ShapeDtypeStruct((n_edges // chunk, ckr, 128), jnp.float32),
        scratch_types=[
            pltpu.VMEM((chunk,), jnp.int32),
            pltpu.VMEM((chunk, D), jnp.float32),
            pltpu.VMEM((ckr, 128), jnp.float32),
            pltpu.SemaphoreType.DMA,
        ],
        compiler_params=_SCP,
    )
    def k(table_hbm, idx_hbm, out_hbm, idx_v, rows_v, packed_v, sem):
        wid = lax.axis_index("s") * _NC + lax.axis_index("c")
        base = wid * per_w

        def body(i, carry):
            off = base + i * chunk
            pltpu.sync_copy(idx_hbm.at[pl.ds(base_edge + off, chunk)], idx_v)
            pltpu.async_copy(table_hbm.at[idx_v], rows_v, sem).wait()

            def pack(j, c2):
                for l in range(8):
                    packed_v[j, pl.ds(l * D, D)] = rows_v[l * ckr + j, :]
                return c2

            lax.fori_loop(0, ckr, pack, 0)
            pltpu.sync_copy(packed_v, out_hbm.at[off // chunk])
            return carry

        lax.fori_loop(0, n_ch, body, 0)

    return k(table, idx)


def _sc_scatter_add(rows_packed, idx, base_edge, n_out, chunk=2000):
    D = D_OUT
    E = rows_packed.shape[0] * chunk
    per_core = E // _NC
    per_w = per_core // _NS
    n_ch = per_w // chunk
    ckr = chunk * D // 128
    rpt = ((n_out // _NS) // 8) * 8
    rem = n_out - rpt * _NS
    out_rows = 1280
    mesh = plsc.VectorSubcoreMesh(core_axis_name="c", subcore_axis_name="s")

    @functools.partial(
        pl.kernel,
        mesh=mesh,
        out_type=jax.ShapeDtypeStruct((_NC, out_rows, 128), jnp.float32),
        scratch_types=[
            pltpu.VMEM((chunk,), jnp.int32),
            pltpu.VMEM((ckr, 128), jnp.float32),
            pltpu.VMEM((chunk, D), jnp.float32),
            pltpu.VMEM((rpt + rem, D), jnp.float32),
            pltpu.VMEM(((rpt + rem) * D // 128, 128), jnp.float32),
            pltpu.VMEM_SHARED((n_out, D), jnp.float32),
            pltpu.SemaphoreType.DMA,
        ],
        compiler_params=_SCP,
    )
    def k(rp_hbm, idx_hbm, zeros_hbm, out_hbm, idx_v, packed_v, rows_v, bounce,
          bpk, accum, sem):
        c = lax.axis_index("c")
        s = lax.axis_index("s")
        base = c * per_core + s * per_w
        zoff = s * rpt

        pltpu.sync_copy(
            zeros_hbm.at[pl.ds(zoff, rpt)],
            accum.at[pl.ds(zoff, rpt)],
        )

        @pl.when(s == _NS - 1)
        def _zero_tail():
            pltpu.sync_copy(
                zeros_hbm.at[pl.ds(rpt * _NS, rem)],
                accum.at[pl.ds(rpt * _NS, rem)],
            )

        plsc.subcore_barrier()

        def body(i, carry):
            off = base + i * chunk
            pltpu.sync_copy(idx_hbm.at[pl.ds(base_edge + off, chunk)], idx_v)
            pltpu.sync_copy(rp_hbm.at[off // chunk], packed_v)

            def unpack(j, c2):
                for l in range(8):
                    rows_v[l * ckr + j, :] = packed_v[j, pl.ds(l * D, D)]
                return c2

            lax.fori_loop(0, ckr, unpack, 0)
            pltpu.sync_copy(rows_v, accum.at[idx_v], add=True)
            return carry

        lax.fori_loop(0, n_ch, body, 0)
        plsc.subcore_barrier()

        def flush(src_off, nrows, dst_row):
            pltpu.sync_copy(accum.at[pl.ds(src_off, nrows)], bounce.at[pl.ds(0, nrows)])

            def pack(j, c2):
                for l in range(8):
                    bpk[j, pl.ds(l * D, D)] = bounce[j * 8 + l, :]
                return c2

            lax.fori_loop(0, nrows * D // 128, pack, 0)
            pltpu.sync_copy(
                bpk.at[pl.ds(0, nrows * D // 128)],
                out_hbm.at[c, pl.ds(dst_row, nrows * D // 128)],
            )

        flush(zoff, rpt, (zoff * D) // 128)

        @pl.when(s == _NS - 1)
        def _tail():
            flush(rpt * _NS, rem, (rpt * _NS * D) // 128)

    zeros = jnp.zeros((n_out, D), jnp.float32)
    return k(rows_packed, idx, zeros)


def _tc_dense(ell, xp, w1n, w2n, block_base=0, block=2000):
    pk = block * D_IN // 128
    sub = block // 8
    grid = xp.shape[0]
    assert xp.shape == (grid, pk, 128)

    R = np.zeros((8, 128, HIDDEN), np.float32)
    U = np.zeros((8, HIDDEN, 128), np.float32)
    for l in range(8):
        for i in range(D_IN):
            for k in range(D_OUT):
                R[l, 16 * l + i, 16 * i + k] = 1.0
                U[l, 16 * i + k, 16 * l + k] = 1.0
    rcat = jnp.asarray(R.reshape(8 * 128, HIDDEN))
    ucat = jnp.asarray(U.reshape(8 * HIDDEN, 128))

    def body(lsh_ref, x_ref, w1_ref, w2_ref, r_ref, u_ref, o_ref):
        lsh = lsh_ref[...].astype(jnp.float32)
        h = jnp.maximum(
            jnp.dot(lsh, w1_ref[...], preferred_element_type=jnp.float32), 0.0
        )
        wt = jnp.dot(h, w2_ref[...], preferred_element_type=jnp.float32)
        wts = wt * lsh[:, 3:4]
        xpb = x_ref[0]
        o = None
        for l in range(8):
            xr = jnp.dot(
                xpb, r_ref[128 * l : 128 * (l + 1), :],
                preferred_element_type=jnp.float32,
            )
            y = xr * wts[sub * l : sub * (l + 1), :]
            t = jnp.dot(
                y, u_ref[HIDDEN * l : HIDDEN * (l + 1), :],
                preferred_element_type=jnp.float32,
            )
            o = t if o is None else o + t
        o_ref[0] = o

    return pl.pallas_call(
        body,
        grid=(grid,),
        in_specs=[
            pl.BlockSpec((block, 4), lambda i: (i + block_base, 0)),
            pl.BlockSpec((1, pk, 128), lambda i: (i, 0, 0)),
            pl.BlockSpec((4, HIDDEN), lambda i: (0, 0)),
            pl.BlockSpec((HIDDEN, HIDDEN), lambda i: (0, 0)),
            pl.BlockSpec((8 * 128, HIDDEN), lambda i: (0, 0)),
            pl.BlockSpec((8 * HIDDEN, 128), lambda i: (0, 0)),
        ],
        out_specs=pl.BlockSpec((1, pk, 128), lambda i: (i, 0, 0)),
        out_shape=jax.ShapeDtypeStruct((grid, pk, 128), jnp.float32),
    )(ell, xp, w1n, w2n, rcat, ucat)


def _tc_combine(pa, pb):
    rows = pa.shape[1]
    blk = 128

    def body(a_ref, b_ref, o_ref):
        o_ref[...] = a_ref[0] + a_ref[1] + b_ref[0] + b_ref[1]

    return pl.pallas_call(
        body,
        grid=(rows // blk,),
        in_specs=[
            pl.BlockSpec((2, blk, 128), lambda i: (0, i, 0)),
            pl.BlockSpec((2, blk, 128), lambda i: (0, i, 0)),
        ],
        out_specs=pl.BlockSpec((blk, 128), lambda i: (i, 0)),
        out_shape=jax.ShapeDtypeStruct((rows, 128), jnp.float32),
    )(pa, pb)


def kernel(edge_src, edge_dst, node_features, edge_sh, edge_length_embedded,
           num_neighbors, W1, W2):
    E = edge_src.shape[0]
    w1n = (W1 * np.float32(1.0 / np.sqrt(3.0))).astype(jnp.float32)
    scale = np.float32(np.sqrt(2.0) / np.sqrt(float(HIDDEN)) / np.sqrt(float(D_IN)))
    w2n = W2 * (scale / jnp.sqrt(jnp.float32(num_neighbors)))

    lsh = jnp.concatenate(
        [edge_length_embedded, edge_sh], axis=1
    ).astype(jnp.bfloat16)
    w1p = jnp.pad(w1n, ((0, 1), (0, 0)))

    Eh = E // 2
    src = edge_src.astype(jnp.int32)
    dst = edge_dst.astype(jnp.int32)
    parts = []
    efps = []
    for hf in range(2):
        xp = _sc_gather(node_features, src, hf * Eh, Eh)
        efps.append(_tc_dense(lsh, xp, w1p, w2n, block_base=hf * (Eh // 2000)))
    for hf in range(2):
        parts.append(_sc_scatter_add(efps[hf], dst, hf * Eh, N_NODES))
    comb = _tc_combine(*parts)
    return comb.reshape(-1)[: N_NODES * D_OUT].reshape(N_NODES, D_OUT)
